# baseline (speedup 1.0000x reference)
import functools

import jax
import jax.numpy as jnp
from jax import lax
from jax.experimental import pallas as pl
from jax.experimental.pallas import tpu as pltpu
from jax.experimental.pallas import tpu_sc as plsc

N_TOKENS = 100000
D = 128
D4 = 32
B, L = 4096, 200
N = B * L

NC, NS, LANES = 2, 16, 16
NW = NC * NS
NPW = N // NW
K = 128
CHUNKS = NPW // K

_EVEN_COLS = tuple(32 * m + i for m in range(4) for i in range(16))
_ODD_COLS = tuple(32 * m + 16 + i for m in range(4) for i in range(16))


_ROWS = 10000


def _tt_body(tbl_ref, we_ref, wo_ref, out_ref):
    tbl = tbl_ref[...]
    ye = jnp.dot(tbl, we_ref[0], preferred_element_type=jnp.float32)
    yo = jnp.dot(tbl, wo_ref[0], preferred_element_type=jnp.float32)
    eb = lax.bitcast_convert_type(
        ye.astype(jnp.bfloat16).astype(jnp.float32), jnp.uint32)
    ob = lax.bitcast_convert_type(
        yo.astype(jnp.bfloat16).astype(jnp.float32), jnp.uint32)
    packed = (eb >> jnp.uint32(16)) | (ob & jnp.uint32(0xFFFF0000))
    out_ref[...] = lax.bitcast_convert_type(packed, jnp.int32)


def _make_tt(big_table, w3e, w3o):
    grid = (3 * N_TOKENS) // _ROWS
    return pl.pallas_call(
        _tt_body,
        grid=(grid,),
        in_specs=[
            pl.BlockSpec((_ROWS, D4), lambda i: (i, 0)),
            pl.BlockSpec((1, D4, D // 2),
                         lambda i: ((i * _ROWS) // N_TOKENS, 0, 0)),
            pl.BlockSpec((1, D4, D // 2),
                         lambda i: ((i * _ROWS) // N_TOKENS, 0, 0)),
        ],
        out_specs=pl.BlockSpec((_ROWS, D // 2), lambda i: (i, 0)),
        out_shape=jax.ShapeDtypeStruct((3 * N_TOKENS, D // 2), jnp.int32),
    )(big_table, w3e, w3o)


def _uc_body(p_ref, w_ref, pb_ref, out_ref):
    out_ref[...] = jnp.dot(p_ref[...], w_ref[...],
                           preferred_element_type=jnp.float32) + pb_ref[...]


def _make_uc(p8, w3v, pb8):
    return pl.pallas_call(
        _uc_body,
        out_shape=jax.ShapeDtypeStruct((8, D), jnp.float32),
    )(p8, w3v, pb8)


def _bf16x2_to_f32(w):
    lo = plsc.bitcast(w << 16, jnp.float32)
    hi = plsc.bitcast(w & jnp.int32(-65536), jnp.float32)
    return lo, hi


def _sc_body(seqb_hbm, tt_hbm, uc_hbm, out_hbm,
             seq0, seq1, ti0, ti1, ai0, ai1, bi0, bi1,
             rt0, rt1, ra0, ra1, rb0, rb1, st0, st1, uc_v,
             gs0, gs1, ss0, ss1):
    seqv = [seq0, seq1]
    tiv = [ti0, ti1]
    aiv = [ai0, ai1]
    biv = [bi0, bi1]
    rtv = [rt0, rt1]
    rav = [ra0, ra1]
    rbv = [rb0, rb1]
    stv = [st0, st1]
    gsem = [gs0, gs1]
    ssem = [ss0, ss1]

    wid = lax.axis_index("s") * NC + lax.axis_index("c")
    c_base = wid * CHUNKS

    pltpu.sync_copy(uc_hbm, uc_v)
    u_rows = [uc_v[0, pl.ds(16 * j, 16)] for j in range(D // 16)]
    c_rows = [uc_v[1, pl.ds(16 * j, 16)] for j in range(D // 16)]

    def load_and_fire(ci, b):
        pltpu.sync_copy(seqb_hbm.at[c_base + ci], seqv[b])
        for g in range(K // LANES):
            s = pl.ds(g * LANES, LANES)
            tiv[b][s] = jnp.clip(seqv[b][0, s].astype(jnp.int32),
                                 0, N_TOKENS - 1)
            aiv[b][s] = jnp.clip(seqv[b][1, s].astype(jnp.int32),
                                 0, N_TOKENS - 1) + N_TOKENS
            biv[b][s] = jnp.clip(seqv[b][2, s].astype(jnp.int32),
                                 0, N_TOKENS - 1) + 2 * N_TOKENS
        pltpu.async_copy(tt_hbm.at[tiv[b]], rtv[b], gsem[b])
        pltpu.async_copy(tt_hbm.at[aiv[b]], rav[b], gsem[b])
        pltpu.async_copy(tt_hbm.at[biv[b]], rbv[b], gsem[b])

    def wait_gathers(b):
        pltpu.make_async_copy(tt_hbm.at[tiv[b]], rtv[b], gsem[b]).wait()
        pltpu.make_async_copy(tt_hbm.at[aiv[b]], rav[b], gsem[b]).wait()
        pltpu.make_async_copy(tt_hbm.at[biv[b]], rbv[b], gsem[b]).wait()

    def compute(b):
        def grp_body(g, carry2):
            vblk = seqv[b][3, pl.ds(g * LANES, LANES)]
            for t in range(LANES):
                k = g * LANES + t
                vk = jnp.broadcast_to(vblk[t], (LANES,))
                for h in range(D // 32):
                    s16 = pl.ds(16 * h, 16)
                    t0, t1 = _bf16x2_to_f32(rtv[b][k, s16])
                    a0, a1 = _bf16x2_to_f32(rav[b][k, s16])
                    b0, b1 = _bf16x2_to_f32(rbv[b][k, s16])
                    stv[b][k, pl.ds(32 * h, 16)] = (
                        t0 + a0 + b0 + vk * u_rows[2 * h] + c_rows[2 * h])
                    stv[b][k, pl.ds(32 * h + 16, 16)] = (
                        t1 + a1 + b1 + vk * u_rows[2 * h + 1]
                        + c_rows[2 * h + 1])
            return carry2

        lax.fori_loop(0, K // LANES, grp_body, 0)

    def out_slice(ci):
        return out_hbm.at[pl.ds((c_base + ci) * K, K)]

    def fire_store(ci, b):
        pltpu.async_copy(stv[b], out_slice(ci), ssem[b])

    def wait_store(ci, b):
        pltpu.make_async_copy(stv[b], out_slice(ci), ssem[b]).wait()

    load_and_fire(0, 0)

    def pair_body(p, carry):
        ci = 2 * p

        @pl.when(ci >= 2)
        def _():
            wait_store(ci - 2, 0)

        load_and_fire(ci + 1, 1)
        wait_gathers(0)
        compute(0)
        fire_store(ci, 0)

        @pl.when(ci >= 1)
        def _():
            wait_store(ci - 1, 1)

        @pl.when(ci + 2 < CHUNKS)
        def _():
            load_and_fire(ci + 2, 0)

        wait_gathers(1)
        compute(1)
        fire_store(ci + 1, 1)
        return carry

    lax.fori_loop(0, CHUNKS // 2, pair_body, 0)
    wait_store(CHUNKS - 2, 0)
    wait_store(CHUNKS - 1, 1)


@functools.cache
def _sc_gather_fn():
    return pl.kernel(
        _sc_body,
        out_type=jax.ShapeDtypeStruct((N, D), jnp.float32),
        mesh=plsc.VectorSubcoreMesh(core_axis_name="c", subcore_axis_name="s",
                                    num_cores=NC, num_subcores=NS),
        compiler_params=pltpu.CompilerParams(needs_layout_passes=False,
                                             use_tc_tiling_on_sc=False),
        scratch_types=[
            pltpu.VMEM((4, K), jnp.float32),
            pltpu.VMEM((4, K), jnp.float32),
            pltpu.VMEM((K,), jnp.int32),
            pltpu.VMEM((K,), jnp.int32),
            pltpu.VMEM((K,), jnp.int32),
            pltpu.VMEM((K,), jnp.int32),
            pltpu.VMEM((K,), jnp.int32),
            pltpu.VMEM((K,), jnp.int32),
            pltpu.VMEM((K, D // 2), jnp.int32),
            pltpu.VMEM((K, D // 2), jnp.int32),
            pltpu.VMEM((K, D // 2), jnp.int32),
            pltpu.VMEM((K, D // 2), jnp.int32),
            pltpu.VMEM((K, D // 2), jnp.int32),
            pltpu.VMEM((K, D // 2), jnp.int32),
            pltpu.VMEM((K, D), jnp.float32),
            pltpu.VMEM((K, D), jnp.float32),
            pltpu.VMEM((8, D), jnp.float32),
            pltpu.SemaphoreType.DMA,
            pltpu.SemaphoreType.DMA,
            pltpu.SemaphoreType.DMA,
            pltpu.SemaphoreType.DMA,
        ],
    )


def kernel(seq, type_emb, node_a_emb, node_b_emb, vp_w, vp_b, proj_w, proj_b):
    big_table = jnp.concatenate([type_emb, node_a_emb, node_b_emb], axis=0)
    we = jnp.asarray(_EVEN_COLS, dtype=jnp.int32)
    wo = jnp.asarray(_ODD_COLS, dtype=jnp.int32)
    w3e = proj_w[: 3 * D4, we].reshape(3, D4, D // 2)
    w3o = proj_w[: 3 * D4, wo].reshape(3, D4, D // 2)
    tt = _make_tt(big_table, w3e, w3o)

    p8 = jnp.zeros((8, D4), jnp.float32).at[0].set(vp_w[0]).at[1].set(vp_b)
    pb8 = jnp.zeros((8, D), jnp.float32).at[1].set(proj_b)
    uc = _make_uc(p8, proj_w[3 * D4:], pb8)

    seqb = seq.reshape(N // K, K, 4).transpose(0, 2, 1)
    out = _sc_gather_fn()(seqb, tt, uc)
    return out.reshape(B, L, D)


# speedup vs baseline: 2.1123x; 2.1123x over previous
import functools

import jax
import jax.numpy as jnp
from jax import lax
from jax.experimental import pallas as pl
from jax.experimental.pallas import tpu as pltpu
from jax.experimental.pallas import tpu_sc as plsc

N_TOKENS = 100000
D = 128
D4 = 32
B, L = 4096, 200
N = B * L

NC, NS, LANES = 2, 16, 16
NW = NC * NS
NPW = N // NW
K = 128
CHUNKS = NPW // K
RING = 4


_ROWS = 10000


def _tt_body(tbl_ref, w_ref, c_ref, out_ref):
    y = jnp.dot(tbl_ref[...], w_ref[0], preferred_element_type=jnp.float32)
    sel = jnp.where(pl.program_id(0) * _ROWS < N_TOKENS, 1.0, 0.0)
    out_ref[...] = y + sel * c_ref[...]


def _make_tt(big_table, w3, c_row):
    grid = (3 * N_TOKENS) // _ROWS
    return pl.pallas_call(
        _tt_body,
        grid=(grid,),
        in_specs=[
            pl.BlockSpec((_ROWS, D4), lambda i: (i, 0)),
            pl.BlockSpec((1, D4, D), lambda i: ((i * _ROWS) // N_TOKENS, 0, 0)),
            pl.BlockSpec((1, D), lambda i: (0, 0)),
        ],
        out_specs=pl.BlockSpec((_ROWS, D), lambda i: (i, 0)),
        out_shape=jax.ShapeDtypeStruct((3 * N_TOKENS, D), jnp.float32),
    )(big_table, w3, c_row)


def _uc_body(p_ref, w_ref, pb_ref, out_ref):
    out_ref[...] = jnp.dot(p_ref[...], w_ref[...],
                           preferred_element_type=jnp.float32) + pb_ref[...]


def _make_uc(p8, w3v, pb8):
    return pl.pallas_call(
        _uc_body,
        out_shape=jax.ShapeDtypeStruct((8, D), jnp.float32),
    )(p8, w3v, pb8)


def _sc_body(seqb_hbm, tt_hbm, uc_hbm, out_hbm,
             seq0, seq1, seq2, seq3,
             ti0, ti1, ti2, ti3, ai0, ai1, ai2, ai3, bi0, bi1, bi2, bi3,
             rt0, rt1, rt2, rt3, uc_v,
             gs0, gs1, gs2, gs3, os0, os1, os2, os3):
    seqv = [seq0, seq1, seq2, seq3]
    tiv = [ti0, ti1, ti2, ti3]
    aiv = [ai0, ai1, ai2, ai3]
    biv = [bi0, bi1, bi2, bi3]
    rtv = [rt0, rt1, rt2, rt3]
    gsem = [gs0, gs1, gs2, gs3]
    osem = [os0, os1, os2, os3]

    wid = lax.axis_index("s") * NC + lax.axis_index("c")
    c_base = wid * CHUNKS

    pltpu.sync_copy(uc_hbm, uc_v)
    u_rows = [uc_v[0, pl.ds(16 * j, 16)] for j in range(D // 16)]

    def load_and_fire(ci, b):
        pltpu.sync_copy(seqb_hbm.at[c_base + ci], seqv[b])
        for g in range(K // LANES):
            s = pl.ds(g * LANES, LANES)
            tiv[b][s] = jnp.clip(seqv[b][0, s].astype(jnp.int32),
                                 0, N_TOKENS - 1)
            aiv[b][s] = jnp.clip(seqv[b][1, s].astype(jnp.int32),
                                 0, N_TOKENS - 1) + N_TOKENS
            biv[b][s] = jnp.clip(seqv[b][2, s].astype(jnp.int32),
                                 0, N_TOKENS - 1) + 2 * N_TOKENS

        def grp_body(g, carry):
            vblk = seqv[b][3, pl.ds(g * LANES, LANES)]
            for t in range(LANES):
                k = g * LANES + t
                vk = jnp.broadcast_to(vblk[t], (LANES,))
                for j in range(D // 16):
                    rtv[b][k, pl.ds(16 * j, 16)] = vk * u_rows[j]
            return carry

        lax.fori_loop(0, K // LANES, grp_body, 0)
        pltpu.async_copy(tt_hbm.at[tiv[b]], rtv[b], gsem[b], add=True)
        pltpu.async_copy(tt_hbm.at[aiv[b]], rtv[b], gsem[b], add=True)
        pltpu.async_copy(tt_hbm.at[biv[b]], rtv[b], gsem[b], add=True)

    def wait_gathers(b):
        pltpu.make_async_copy(tt_hbm.at[tiv[b]], rtv[b], gsem[b]).wait()
        pltpu.make_async_copy(tt_hbm.at[aiv[b]], rtv[b], gsem[b]).wait()
        pltpu.make_async_copy(tt_hbm.at[biv[b]], rtv[b], gsem[b]).wait()

    def out_slice(ci):
        return out_hbm.at[pl.ds((c_base + ci) * K, K)]

    def fire_out(ci, b):
        pltpu.async_copy(rtv[b], out_slice(ci), osem[b])

    def wait_out(ci, b):
        pltpu.make_async_copy(rtv[b], out_slice(ci), osem[b]).wait()

    for b in range(RING - 1):
        load_and_fire(b, b)

    def ring_body(p, carry):
        for q in range(RING):
            n = RING * p + q
            wait_gathers(q)
            fire_out(n, q)
            m = n + RING - 1
            bm = (q + RING - 1) % RING

            @pl.when(m < CHUNKS)
            def _():
                @pl.when(m >= RING)
                def _():
                    wait_out(m - RING, bm)
                load_and_fire(m, bm)
        return carry

    lax.fori_loop(0, CHUNKS // RING, ring_body, 0)
    for q in range(RING):
        wait_out(CHUNKS - RING + q, q)


@functools.cache
def _sc_gather_fn():
    return pl.kernel(
        _sc_body,
        out_type=jax.ShapeDtypeStruct((N, D), jnp.float32),
        mesh=plsc.VectorSubcoreMesh(core_axis_name="c", subcore_axis_name="s",
                                    num_cores=NC, num_subcores=NS),
        compiler_params=pltpu.CompilerParams(needs_layout_passes=False),
        scratch_types=(
            [pltpu.VMEM((4, K), jnp.float32) for _ in range(RING)]
            + [pltpu.VMEM((K,), jnp.int32) for _ in range(3 * RING)]
            + [pltpu.VMEM((K, D), jnp.float32) for _ in range(RING)]
            + [pltpu.VMEM((8, D), jnp.float32)]
            + [pltpu.SemaphoreType.DMA for _ in range(2 * RING)]
        ),
    )


def kernel(seq, type_emb, node_a_emb, node_b_emb, vp_w, vp_b, proj_w, proj_b):
    p8 = jnp.zeros((8, D4), jnp.float32).at[0].set(vp_w[0]).at[1].set(vp_b)
    pb8 = jnp.zeros((8, D), jnp.float32).at[1].set(proj_b)
    uc = _make_uc(p8, proj_w[3 * D4:], pb8)

    big_table = jnp.concatenate([type_emb, node_a_emb, node_b_emb], axis=0)
    w3 = proj_w[: 3 * D4].reshape(3, D4, D)
    tt = _make_tt(big_table, w3, uc[1:2])

    seqb = seq.reshape(N // K, K, 4).transpose(0, 2, 1)
    out = _sc_gather_fn()(seqb, tt, uc)
    return out.reshape(B, L, D)
